# baseline (speedup 1.0000x reference)
import functools

import jax
import jax.numpy as jnp
from jax import lax
from jax.experimental import pallas as pl
from jax.experimental.pallas import tpu as pltpu
from jax.experimental.pallas import tpu_sc as plsc

VOCAB = 1000000
EMB = 64
B = 16384
L = 20

NC = 2
NS = 16
NW = NC * NS
BW = B // NW
C = 32
NCH = BW // C
PAIRS = C * L
IDXW = 128
VROWS = BW // IDXW
UROWS = PAIRS // IDXW
ECH = EMB // 16

_mesh = plsc.VectorSubcoreMesh(core_axis_name="c", subcore_axis_name="s")


@functools.partial(
    pl.kernel,
    mesh=_mesh,
    out_type=jax.ShapeDtypeStruct((B * L,), jnp.float32),
    scratch_types=[
        pltpu.VMEM((VROWS, IDXW), jnp.int32),
        pltpu.VMEM((UROWS, IDXW), jnp.int32),
        pltpu.VMEM((BW, EMB), jnp.float32),
        pltpu.VMEM((PAIRS, EMB), jnp.float32),
        pltpu.VMEM((BW * L,), jnp.float32),
        pltpu.VMEM((80 * 16,), jnp.float32),
        pltpu.SemaphoreType.DMA,
    ],
    compiler_params=pltpu.CompilerParams(
        needs_layout_passes=False, use_tc_tiling_on_sc=False),
)
def _sc_kernel(cidx_hbm, uidx_hbm, v_hbm, u_hbm, out_hbm,
               vi_v, ui_v, v_rows, u_rows, out_v, psum, sem):
    wid = lax.axis_index("s") * NC + lax.axis_index("c")

    for j in range(VROWS):
        pltpu.sync_copy(cidx_hbm.at[pl.ds(wid * BW + j * IDXW, IDXW)],
                        vi_v.at[j])
    vcps = [
        pltpu.async_copy(v_hbm.at[vi_v.at[j]],
                         v_rows.at[pl.ds(j * IDXW, IDXW)], sem)
        for j in range(VROWS)
    ]
    for cp in vcps:
        cp.wait()

    def chunk_body(ch, carry):
        base = (wid * NCH + ch) * PAIRS
        for j in range(UROWS):
            pltpu.sync_copy(uidx_hbm.at[pl.ds(base + j * IDXW, IDXW)],
                            ui_v.at[j])
        ucps = [
            pltpu.async_copy(u_hbm.at[ui_v.at[j]],
                             u_rows.at[pl.ds(j * IDXW, IDXW)], sem)
            for j in range(UROWS)
        ]
        for cp in ucps:
            cp.wait()

        lane = lax.iota(jnp.int32, 16)

        def group_body(g, carry2):
            for bb in range(4):
                cb = ch * C + g * 4 + bb
                vv = [v_rows[cb, pl.ds(e * 16, 16)] for e in range(ECH)]
                for l in range(L):
                    pf = bb * L + l
                    p = g * 80 + pf
                    acc = u_rows[p, pl.ds(0, 16)] * vv[0]
                    for e in range(1, ECH):
                        acc = acc + u_rows[p, pl.ds(e * 16, 16)] * vv[e]
                    psum[pl.ds(pf * 16, 16)] = acc
            out_base = ch * PAIRS + g * 80
            for gg in range(5):
                rows = (lane + gg * 16) * 16
                r = plsc.load_gather(psum, [rows])
                for j in range(1, 16):
                    r = r + plsc.load_gather(psum, [rows + j])
                out_v[pl.ds(out_base + gg * 16, 16)] = r
            return carry2

        lax.fori_loop(0, C // 4, group_body, 0)
        return carry

    lax.fori_loop(0, NCH, chunk_body, 0)
    pltpu.sync_copy(out_v, out_hbm.at[pl.ds(wid * BW * L, BW * L)])


def kernel(center, contexts_and_negatives, embed_v, embed_u):
    cidx = center.reshape(B)
    uidx = contexts_and_negatives.reshape(B * L)
    out = _sc_kernel(cidx, uidx, embed_v, embed_u)
    return out.reshape(B, 1, L)
